# Initial kernel scaffold; baseline (speedup 1.0000x reference)
import jax
import jax.numpy as jnp
from jax.experimental import pallas as pl


def kernel(x):
    raise NotImplementedError("write your pallas kernel here")


# speedup vs baseline: 1.2117x; 1.2117x over previous
import functools

import jax
import jax.numpy as jnp
from jax import lax
from jax.experimental import pallas as pl
from jax.experimental.pallas import tpu as pltpu

_LANE = 128


def _extrema_scatter_kernel(win_ref, o_ref, *, pool_size: int, width: int,
                            region: int):
    win = win_ref[...]
    ts, c_dim, pp = win.shape
    awin = jnp.abs(win)
    jpos = lax.broadcasted_iota(jnp.int32, awin.shape, 2)
    m = jnp.max(awin, axis=-1, keepdims=True)
    cand = jnp.where(awin == m, jpos, jnp.int32(pp))
    jidx = jnp.min(cand, axis=-1, keepdims=True)
    idx = (jidx // pool_size) * width + (jidx % pool_size)
    hit0 = jpos == jidx
    val = jnp.sum(jnp.where(hit0, win[:, :1, :], 0.0),
                  axis=-1, keepdims=True).astype(o_ref.dtype)
    col = lax.broadcasted_iota(jnp.int32, (1, region), 1)
    acc = jnp.zeros((ts, region), o_ref.dtype)
    for c in range(c_dim):
        acc = jnp.where(col == idx[:, c, :], val[:, c, :], acc)
    o_ref[:, :region] = acc
    o_ref[:, region:] = jnp.zeros((ts, o_ref.shape[1] - region), o_ref.dtype)


def _extrema_pool_indices_2d(x, pool_size: int):
    N, C, H, W = x.shape
    HW = H * W
    pp = pool_size * pool_size
    itemsize = x.dtype.itemsize

    win = x[:, :, :pool_size, :pool_size].reshape(N, C, pp)

    region = -(-((pool_size - 1) * W + pool_size) // _LANE) * _LANE
    region = min(region, C * HW)

    row = C * HW
    ts = max(8, min(N, (8 * 1024 * 1024) // (row * itemsize) // 8 * 8))

    out2 = pl.pallas_call(
        functools.partial(_extrema_scatter_kernel, pool_size=pool_size,
                          width=W, region=region),
        out_shape=jax.ShapeDtypeStruct((N, row), x.dtype),
        grid=(pl.cdiv(N, ts),),
        in_specs=[pl.BlockSpec((ts, C, pp), lambda i: (i, 0, 0))],
        out_specs=pl.BlockSpec((ts, row), lambda i: (i, 0)),
        compiler_params=pltpu.CompilerParams(
            dimension_semantics=("parallel",),
            vmem_limit_bytes=64 * 1024 * 1024,
        ),
        cost_estimate=pl.CostEstimate(
            flops=8 * N * C * pp + 2 * N * region,
            transcendentals=0,
            bytes_accessed=(N * row + N * C * pp) * itemsize,
        ),
    )(win)
    return out2.reshape(N, C, H, W)


def kernel(x):
    return _extrema_pool_indices_2d(x, 4)
